# Initial kernel scaffold; baseline (speedup 1.0000x reference)
import jax
import jax.numpy as jnp
from jax.experimental import pallas as pl


def kernel(xyz, features, W1, b1, W2, b2, W3, b3):
    raise NotImplementedError("write your pallas kernel here")


# speedup vs baseline: 10.3964x; 10.3964x over previous
import functools

import jax
import jax.numpy as jnp
import numpy as np
from jax import lax
from jax.experimental import pallas as pl
from jax.experimental.pallas import tpu as pltpu
from jax.experimental.pallas import tpu_sc as plsc

_RATIO = 0.25
_RADIUS = 0.2
_NS = 32
_B = 4
_N = 4096
_S = 1024


def _fps_kernel(x_ref, y_ref, z_ref, xc_ref, yc_ref, zc_ref):
    x = x_ref[...]
    y = y_ref[...]
    z = z_ref[...]
    iota = (lax.broadcasted_iota(jnp.int32, (_B, 32, 128), 1) * 128
            + lax.broadcasted_iota(jnp.int32, (_B, 32, 128), 2))

    def body(i, carry):
        dists, far = carry
        oh = iota == far
        cx = jnp.sum(jnp.where(oh, x, 0.0), axis=(1, 2))
        cy = jnp.sum(jnp.where(oh, y, 0.0), axis=(1, 2))
        cz = jnp.sum(jnp.where(oh, z, 0.0), axis=(1, 2))
        xc_ref[pl.ds(i, 1), :] = cx.reshape(1, _B)
        yc_ref[pl.ds(i, 1), :] = cy.reshape(1, _B)
        zc_ref[pl.ds(i, 1), :] = cz.reshape(1, _B)
        dx = x - cx.reshape(_B, 1, 1)
        dy = y - cy.reshape(_B, 1, 1)
        dz = z - cz.reshape(_B, 1, 1)
        d = dx * dx + dy * dy + dz * dz
        dists = jnp.minimum(dists, d)
        m = jnp.max(dists, axis=(1, 2)).reshape(_B, 1, 1)
        cand = jnp.where(dists == m, iota, jnp.int32(1 << 30))
        far = jnp.min(cand, axis=(1, 2)).reshape(_B, 1, 1)
        return dists, far

    dists0 = jnp.full((_B, 32, 128), 1e10, jnp.float32)
    far0 = jnp.zeros((_B, 1, 1), jnp.int32)
    lax.fori_loop(0, _S, body, (dists0, far0))


_SBLK = 256


def _mask_kernel(a_ref, bt_ref, mask_ref):
    a = a_ref[...].reshape(_SBLK, 3)
    bt = bt_ref[...].reshape(3, _N)
    sumA = jnp.sum(a * a, axis=-1, keepdims=True)
    sumB = jnp.sum(bt * bt, axis=0, keepdims=True)
    E = jnp.dot(a, bt, preferred_element_type=jnp.float32)
    sqr = (sumA + sumB) - 2.0 * E
    keep = jnp.logical_not(sqr > jnp.float32(_RADIUS ** 2))
    mask_ref[...] = keep.astype(jnp.int32).reshape(1, _SBLK, _N)


def _run_mask(new_xyz, xyz):
    return pl.pallas_call(
        _mask_kernel,
        grid=(_B, _S // _SBLK),
        in_specs=[
            pl.BlockSpec((1, _SBLK, 3), lambda b, sb: (b, sb, 0)),
            pl.BlockSpec((1, 3, _N), lambda b, sb: (b, 0, 0)),
        ],
        out_specs=pl.BlockSpec((1, _SBLK, _N), lambda b, sb: (b, sb, 0)),
        out_shape=jax.ShapeDtypeStruct((_B, _S, _N), jnp.int32),
    )(new_xyz, xyz)


_NCHUNK = _N // 16
_IDXPAD = 48
_RW = 128 * _NS


def _sc_select_body(mask_hbm, xyz_hbm, c_hbm, out_hbm,
                    maskbuf, xb, yb, zb, cxb, cyb, czb, idxb, relbuf):
    wid = lax.axis_index("s") * 2 + lax.axis_index("c")
    b = wid // 8
    s0 = (wid % 8) * 128

    pltpu.sync_copy(xyz_hbm.at[pl.ds((b * 3 + 0) * _N, _N)], xb)
    pltpu.sync_copy(xyz_hbm.at[pl.ds((b * 3 + 1) * _N, _N)], yb)
    pltpu.sync_copy(xyz_hbm.at[pl.ds((b * 3 + 2) * _N, _N)], zb)
    pltpu.sync_copy(c_hbm.at[pl.ds((b * 3 + 0) * _S + s0, 128)], cxb)
    pltpu.sync_copy(c_hbm.at[pl.ds((b * 3 + 1) * _S + s0, 128)], cyb)
    pltpu.sync_copy(c_hbm.at[pl.ds((b * 3 + 2) * _S + s0, 128)], czb)

    lane = lax.broadcasted_iota(jnp.int32, (16,), 0)

    def row_body(r, carry):
        row = b * _S + s0 + r
        pltpu.sync_copy(mask_hbm.at[pl.ds(row * _N, _N)], maskbuf)

        def step(ch, cnt):
            mvec = maskbuf[pl.ds(ch * 16, 16)]
            sel = jnp.logical_and(mvec > 0, cnt < _NS)
            idxv = lane + ch * 16
            pfx = plsc.cumsum(sel.astype(jnp.int32))
            pos = jnp.where(sel, cnt + pfx - 1, _IDXPAD - 1)
            plsc.store_scatter(idxb, [pos], idxv)
            pc = plsc.all_reduce_population_count(sel)
            return cnt + jnp.max(pc)

        cnt = lax.fori_loop(0, _NCHUNK, step, jnp.int32(0))

        zero16 = jnp.zeros((16,), jnp.int32)
        first = plsc.load_gather(idxb, [zero16])
        ridx = jnp.full((16,), r, jnp.int32)
        cxv = plsc.load_gather(cxb, [ridx])
        cyv = plsc.load_gather(cyb, [ridx])
        czv = plsc.load_gather(czb, [ridx])
        for h in range(2):
            vh = idxb[pl.ds(h * 16, 16)]
            valid = (lane + h * 16) < cnt
            gi = jnp.where(valid, vh, first)
            off = r * _NS + h * 16
            relbuf[pl.ds(off, 16)] = plsc.load_gather(xb, [gi]) - cxv
            relbuf[pl.ds(_RW + off, 16)] = plsc.load_gather(yb, [gi]) - cyv
            relbuf[pl.ds(2 * _RW + off, 16)] = plsc.load_gather(zb, [gi]) - czv
        return carry

    lax.fori_loop(0, 128, row_body, jnp.int32(0))

    for c in range(3):
        pltpu.sync_copy(relbuf.at[pl.ds(c * _RW, _RW)],
                        out_hbm.at[pl.ds(((b * 3 + c) * _S + s0) * _NS, _RW)])


def _run_sc_select(mask_flat, xyz, cnew):
    f = functools.partial(
        pl.kernel,
        out_type=jax.ShapeDtypeStruct((_B * 3 * _S * _NS,), jnp.float32),
        mesh=plsc.VectorSubcoreMesh(core_axis_name="c", subcore_axis_name="s"),
        compiler_params=pltpu.CompilerParams(needs_layout_passes=False),
        scratch_types=[
            pltpu.VMEM((_N,), jnp.int32),
            pltpu.VMEM((_N,), jnp.float32),
            pltpu.VMEM((_N,), jnp.float32),
            pltpu.VMEM((_N,), jnp.float32),
            pltpu.VMEM((128,), jnp.float32),
            pltpu.VMEM((128,), jnp.float32),
            pltpu.VMEM((128,), jnp.float32),
            pltpu.VMEM((_IDXPAD,), jnp.int32),
            pltpu.VMEM((3 * _RW,), jnp.float32),
        ],
    )(_sc_select_body)
    out = f(mask_flat.reshape(-1), xyz.reshape(-1), cnew.reshape(-1))
    return out.reshape(_B, 3, _S * _NS)


def _run_fps(xyz):
    x = xyz[:, 0, :].reshape(_B, 32, 128)
    y = xyz[:, 1, :].reshape(_B, 32, 128)
    z = xyz[:, 2, :].reshape(_B, 32, 128)
    out_sh = jax.ShapeDtypeStruct((_S, _B), jnp.float32)
    xc, yc, zc = pl.pallas_call(
        _fps_kernel,
        out_shape=(out_sh, out_sh, out_sh),
    )(x, y, z)
    return xc, yc, zc


_MBLK = 8192


def _mlp_kernel(rel_ref, w1_ref, w2_ref, w3_ref, b1_ref, b2_ref, b3_ref, out_ref):
    x = rel_ref[...].reshape(3, _MBLK)
    w1 = w1_ref[...]
    h = ((w1[:, 0:1] * x[0:1, :] + w1[:, 1:2] * x[1:2, :])
         + w1[:, 2:3] * x[2:3, :])
    h = jnp.maximum(h + b1_ref[...], 0.0)
    h = jnp.dot(w2_ref[...], h, preferred_element_type=jnp.float32)
    h = jnp.maximum(h + b2_ref[...], 0.0)
    z = jnp.dot(w3_ref[...], h, preferred_element_type=jnp.float32)
    m = jnp.max(z.reshape(128, _MBLK // _NS, _NS), axis=2)
    out_ref[...] = jnp.maximum(m + b3_ref[...], 0.0).reshape(1, 128, _MBLK // _NS)


def _run_mlp(rel, w1, w2, w3, b1, b2, b3):
    return pl.pallas_call(
        _mlp_kernel,
        grid=(_B, (_S * _NS) // _MBLK),
        in_specs=[
            pl.BlockSpec((1, 3, _MBLK), lambda b, mb: (b, 0, mb)),
            pl.BlockSpec((64, 3), lambda b, mb: (0, 0)),
            pl.BlockSpec((64, 64), lambda b, mb: (0, 0)),
            pl.BlockSpec((128, 64), lambda b, mb: (0, 0)),
            pl.BlockSpec((64, 1), lambda b, mb: (0, 0)),
            pl.BlockSpec((64, 1), lambda b, mb: (0, 0)),
            pl.BlockSpec((128, 1), lambda b, mb: (0, 0)),
        ],
        out_specs=pl.BlockSpec((1, 128, _MBLK // _NS), lambda b, mb: (b, 0, mb)),
        out_shape=jax.ShapeDtypeStruct((_B, 128, _S), jnp.float32),
    )(rel, w1, w2, w3, b1, b2, b3)


def kernel(xyz, features, W1, b1, W2, b2, W3, b3):
    xc, yc, zc = _run_fps(xyz)
    new_xyz = jnp.stack([xc.T, yc.T, zc.T], axis=-1)

    mask = _run_mask(new_xyz, xyz)
    new_xyz_out = jnp.transpose(new_xyz, (0, 2, 1))
    rel = _run_sc_select(mask, xyz, new_xyz_out)
    new_points_out = _run_mlp(rel, W1, W2, W3,
                              b1.reshape(64, 1), b2.reshape(64, 1),
                              b3.reshape(128, 1))
    return (new_xyz_out, new_points_out)


# speedup vs baseline: 13.6667x; 1.3146x over previous
import functools

import jax
import jax.numpy as jnp
import numpy as np
from jax import lax
from jax.experimental import pallas as pl
from jax.experimental.pallas import tpu as pltpu
from jax.experimental.pallas import tpu_sc as plsc

_RATIO = 0.25
_RADIUS = 0.2
_NS = 32
_B = 4
_N = 4096
_S = 1024


def _fps_kernel(x_ref, y_ref, z_ref, xc_ref, yc_ref, zc_ref):
    x = x_ref[...]
    y = y_ref[...]
    z = z_ref[...]
    iota = (lax.broadcasted_iota(jnp.int32, (_B, 32, 128), 1) * 128
            + lax.broadcasted_iota(jnp.int32, (_B, 32, 128), 2))

    def body(i, carry):
        dists, far = carry
        oh = iota == far
        cx = jnp.sum(jnp.where(oh, x, 0.0), axis=(1, 2))
        cy = jnp.sum(jnp.where(oh, y, 0.0), axis=(1, 2))
        cz = jnp.sum(jnp.where(oh, z, 0.0), axis=(1, 2))
        xc_ref[pl.ds(i, 1), :] = cx.reshape(1, _B)
        yc_ref[pl.ds(i, 1), :] = cy.reshape(1, _B)
        zc_ref[pl.ds(i, 1), :] = cz.reshape(1, _B)
        dx = x - cx.reshape(_B, 1, 1)
        dy = y - cy.reshape(_B, 1, 1)
        dz = z - cz.reshape(_B, 1, 1)
        d = dx * dx + dy * dy + dz * dz
        dists = jnp.minimum(dists, d)
        m = jnp.max(dists, axis=(1, 2)).reshape(_B, 1, 1)
        cand = jnp.where(dists == m, iota, jnp.int32(1 << 30))
        far = jnp.min(cand, axis=(1, 2)).reshape(_B, 1, 1)
        return dists, far

    dists0 = jnp.full((_B, 32, 128), 1e10, jnp.float32)
    far0 = jnp.zeros((_B, 1, 1), jnp.int32)
    lax.fori_loop(0, _S, body, (dists0, far0))


_SBLK = 256


def _mask_kernel(a_ref, bt_ref, mask_ref):
    a = a_ref[...].reshape(_SBLK, 3)
    bt = bt_ref[...].reshape(3, _N)
    sumA = jnp.sum(a * a, axis=-1, keepdims=True)
    sumB = jnp.sum(bt * bt, axis=0, keepdims=True)
    E = jnp.dot(a, bt, preferred_element_type=jnp.float32)
    sqr = (sumA + sumB) - 2.0 * E
    keep = jnp.logical_not(sqr > jnp.float32(_RADIUS ** 2))
    mask_ref[...] = keep.astype(jnp.int32).reshape(1, _SBLK, _N)


def _run_mask(new_xyz, xyz):
    return pl.pallas_call(
        _mask_kernel,
        grid=(_B, _S // _SBLK),
        in_specs=[
            pl.BlockSpec((1, _SBLK, 3), lambda b, sb: (b, sb, 0)),
            pl.BlockSpec((1, 3, _N), lambda b, sb: (b, 0, 0)),
        ],
        out_specs=pl.BlockSpec((1, _SBLK, _N), lambda b, sb: (b, sb, 0)),
        out_shape=jax.ShapeDtypeStruct((_B, _S, _N), jnp.int32),
    )(new_xyz, xyz)


_NCHUNK = _N // 16
_IDXPAD = 48
_RW = 128 * _NS


def _sc_select_body(mask_hbm, xyz_hbm, c_hbm, out_hbm,
                    maskbuf, xb, yb, zb, cxb, cyb, czb, idxb, relbuf, cntb):
    wid = lax.axis_index("s") * 2 + lax.axis_index("c")
    b = wid // 8
    s0 = (wid % 8) * 128

    pltpu.sync_copy(xyz_hbm.at[pl.ds((b * 3 + 0) * _N, _N)], xb)
    pltpu.sync_copy(xyz_hbm.at[pl.ds((b * 3 + 1) * _N, _N)], yb)
    pltpu.sync_copy(xyz_hbm.at[pl.ds((b * 3 + 2) * _N, _N)], zb)
    pltpu.sync_copy(c_hbm.at[pl.ds((b * 3 + 0) * _S + s0, 128)], cxb)
    pltpu.sync_copy(c_hbm.at[pl.ds((b * 3 + 1) * _S + s0, 128)], cyb)
    pltpu.sync_copy(c_hbm.at[pl.ds((b * 3 + 2) * _S + s0, 128)], czb)

    lane = lax.broadcasted_iota(jnp.int32, (16,), 0)

    def row_body(r, carry):
        row = b * _S + s0 + r
        pltpu.sync_copy(mask_hbm.at[pl.ds(row * _N, _N)], maskbuf)

        ns_vec = jnp.full((16,), _NS, jnp.int32)

        def group(g, cntv):
            @pl.when(jnp.max(cntv) < _NS)
            def _():
                def step(ch, cv):
                    mvec = maskbuf[pl.ds(ch * 16, 16)]
                    sel = jnp.logical_and(mvec > 0, cv < ns_vec)
                    idxv = lane + ch * 16
                    pfx = plsc.cumsum(sel.astype(jnp.int32))
                    pos = jnp.where(sel, cv + pfx - 1, _IDXPAD - 1)
                    plsc.store_scatter(idxb, [pos], idxv)
                    return cv + plsc.all_reduce_population_count(sel)
                cntb[...] = lax.fori_loop(g * 16, g * 16 + 16, step, cntv)
            return cntb[...]

        cntv = lax.fori_loop(0, _NCHUNK // 16, group,
                             jnp.zeros((16,), jnp.int32))
        cnt = cntv

        zero16 = jnp.zeros((16,), jnp.int32)
        first = plsc.load_gather(idxb, [zero16])
        ridx = jnp.full((16,), r, jnp.int32)
        cxv = plsc.load_gather(cxb, [ridx])
        cyv = plsc.load_gather(cyb, [ridx])
        czv = plsc.load_gather(czb, [ridx])
        for h in range(2):
            vh = idxb[pl.ds(h * 16, 16)]
            valid = (lane + h * 16) < cnt
            gi = jnp.where(valid, vh, first)
            off = r * _NS + h * 16
            relbuf[pl.ds(off, 16)] = plsc.load_gather(xb, [gi]) - cxv
            relbuf[pl.ds(_RW + off, 16)] = plsc.load_gather(yb, [gi]) - cyv
            relbuf[pl.ds(2 * _RW + off, 16)] = plsc.load_gather(zb, [gi]) - czv
        return carry

    lax.fori_loop(0, 128, row_body, jnp.int32(0))

    for c in range(3):
        pltpu.sync_copy(relbuf.at[pl.ds(c * _RW, _RW)],
                        out_hbm.at[pl.ds(((b * 3 + c) * _S + s0) * _NS, _RW)])


def _run_sc_select(mask_flat, xyz, cnew):
    f = functools.partial(
        pl.kernel,
        out_type=jax.ShapeDtypeStruct((_B * 3 * _S * _NS,), jnp.float32),
        mesh=plsc.VectorSubcoreMesh(core_axis_name="c", subcore_axis_name="s"),
        compiler_params=pltpu.CompilerParams(needs_layout_passes=False),
        scratch_types=[
            pltpu.VMEM((_N,), jnp.int32),
            pltpu.VMEM((_N,), jnp.float32),
            pltpu.VMEM((_N,), jnp.float32),
            pltpu.VMEM((_N,), jnp.float32),
            pltpu.VMEM((128,), jnp.float32),
            pltpu.VMEM((128,), jnp.float32),
            pltpu.VMEM((128,), jnp.float32),
            pltpu.VMEM((_IDXPAD,), jnp.int32),
            pltpu.VMEM((3 * _RW,), jnp.float32),
            pltpu.VMEM((16,), jnp.int32),
        ],
    )(_sc_select_body)
    out = f(mask_flat.reshape(-1), xyz.reshape(-1), cnew.reshape(-1))
    return out.reshape(_B, 3, _S * _NS)


def _run_fps(xyz):
    x = xyz[:, 0, :].reshape(_B, 32, 128)
    y = xyz[:, 1, :].reshape(_B, 32, 128)
    z = xyz[:, 2, :].reshape(_B, 32, 128)
    out_sh = jax.ShapeDtypeStruct((_S, _B), jnp.float32)
    xc, yc, zc = pl.pallas_call(
        _fps_kernel,
        out_shape=(out_sh, out_sh, out_sh),
    )(x, y, z)
    return xc, yc, zc


_MBLK = 8192


def _mlp_kernel(rel_ref, w1_ref, w2_ref, w3_ref, b1_ref, b2_ref, b3_ref, out_ref):
    x = rel_ref[...].reshape(3, _MBLK)
    w1 = w1_ref[...]
    h = ((w1[:, 0:1] * x[0:1, :] + w1[:, 1:2] * x[1:2, :])
         + w1[:, 2:3] * x[2:3, :])
    h = jnp.maximum(h + b1_ref[...], 0.0)
    h = jnp.dot(w2_ref[...], h, preferred_element_type=jnp.float32)
    h = jnp.maximum(h + b2_ref[...], 0.0)
    z = jnp.dot(w3_ref[...], h, preferred_element_type=jnp.float32)
    m = jnp.max(z.reshape(128, _MBLK // _NS, _NS), axis=2)
    out_ref[...] = jnp.maximum(m + b3_ref[...], 0.0).reshape(1, 128, _MBLK // _NS)


def _run_mlp(rel, w1, w2, w3, b1, b2, b3):
    return pl.pallas_call(
        _mlp_kernel,
        grid=(_B, (_S * _NS) // _MBLK),
        in_specs=[
            pl.BlockSpec((1, 3, _MBLK), lambda b, mb: (b, 0, mb)),
            pl.BlockSpec((64, 3), lambda b, mb: (0, 0)),
            pl.BlockSpec((64, 64), lambda b, mb: (0, 0)),
            pl.BlockSpec((128, 64), lambda b, mb: (0, 0)),
            pl.BlockSpec((64, 1), lambda b, mb: (0, 0)),
            pl.BlockSpec((64, 1), lambda b, mb: (0, 0)),
            pl.BlockSpec((128, 1), lambda b, mb: (0, 0)),
        ],
        out_specs=pl.BlockSpec((1, 128, _MBLK // _NS), lambda b, mb: (b, 0, mb)),
        out_shape=jax.ShapeDtypeStruct((_B, 128, _S), jnp.float32),
    )(rel, w1, w2, w3, b1, b2, b3)


def kernel(xyz, features, W1, b1, W2, b2, W3, b3):
    xc, yc, zc = _run_fps(xyz)
    new_xyz = jnp.stack([xc.T, yc.T, zc.T], axis=-1)

    mask = _run_mask(new_xyz, xyz)
    new_xyz_out = jnp.transpose(new_xyz, (0, 2, 1))
    rel = _run_sc_select(mask, xyz, new_xyz_out)
    new_points_out = _run_mlp(rel, W1, W2, W3,
                              b1.reshape(64, 1), b2.reshape(64, 1),
                              b3.reshape(128, 1))
    return (new_xyz_out, new_points_out)
